# baseline (speedup 1.0000x reference)
import functools
import math

import jax
import jax.numpy as jnp
from jax import lax
from jax.experimental import pallas as pl
from jax.experimental.pallas import tpu as pltpu
from jax.experimental.pallas import tpu_sc as plsc

N = 10000
E = 160000
G = 256
H = 16
AF = 128
BF = 16
L = 3

NR = N // 8
E8 = E // 8
N_PAD = 10240
NRP = N_PAD // 8
STRIPE = N_PAD // 16
PSTRIPE = NRP // 16

NC = 2
NS = 16
NW = 25
EW = E // NW
CH = 128
NCH = EW // CH
FG = 5
NG = NCH // FG

_SQRT2 = math.sqrt(2.0)


def _gelu(x):
    return 0.5 * x * (1.0 + lax.erf(x / _SQRT2))


_MESH = plsc.VectorSubcoreMesh(core_axis_name="c", subcore_axis_name="s",
                               num_cores=NC, num_subcores=NS)
_SC_PARAMS = pltpu.CompilerParams(use_tc_tiling_on_sc=False)


@functools.partial(
    pl.kernel,
    out_type=jax.ShapeDtypeStruct((NW, NCH, CH, H), jnp.float32),
    mesh=_MESH,
    compiler_params=_SC_PARAMS,
    scratch_types=[
        pltpu.VMEM_SHARED((N_PAD, H), jnp.float32),
        pltpu.VMEM((NCH, CH), jnp.int32),
        pltpu.VMEM((2, FG, CH, H), jnp.float32),
        pltpu.SemaphoreType.DMA,
        pltpu.SemaphoreType.DMA,
    ],
)
def _sc_gather(hp_hbm, src_hbm, out_hbm, h_sh, idx_v, gb_v, sem, sem2):
    c = lax.axis_index("c")
    s = lax.axis_index("s")
    w = c * NS + s
    pltpu.sync_copy(hp_hbm.at[pl.ds(s * STRIPE, STRIPE)],
                    h_sh.at[pl.ds(s * STRIPE, STRIPE)])
    plsc.subcore_barrier()

    @pl.when(w < NW)
    def _():
        pltpu.sync_copy(src_hbm.at[w], idx_v)

        def step(g, carry):
            @pl.when(g < NG)
            def _():
                for u in range(FG):
                    pltpu.async_copy(
                        h_sh.at[idx_v.at[g * FG + u]],
                        gb_v.at[g % 2].at[u], sem)

            @pl.when(g > 0)
            def _():
                p = g - 1
                for u in range(FG):
                    pltpu.make_async_copy(
                        h_sh.at[idx_v.at[p * FG + u]],
                        gb_v.at[p % 2].at[u], sem).wait()
                pltpu.async_copy(
                    gb_v.at[p % 2],
                    out_hbm.at[w].at[pl.ds(p * FG, FG)], sem2).wait()

            return carry

        lax.fori_loop(0, NG + 1, step, 0)


@functools.partial(
    pl.kernel,
    out_type=(jax.ShapeDtypeStruct((NW, NCH, CH, H), jnp.float32),
              jax.ShapeDtypeStruct((NC, N_PAD, H), jnp.float32)),
    mesh=_MESH,
    compiler_params=_SC_PARAMS,
    scratch_types=[
        pltpu.VMEM_SHARED((N_PAD, H), jnp.float32),
        pltpu.VMEM_SHARED((N_PAD, H), jnp.float32),
        pltpu.VMEM((NCH, CH), jnp.int32),
        pltpu.VMEM((NCH, CH), jnp.int32),
        pltpu.VMEM((2, FG, CH, H), jnp.float32),
        pltpu.VMEM((CH, H), jnp.float32),
        pltpu.VMEM((128, H), jnp.float32),
        pltpu.SemaphoreType.DMA,
        pltpu.SemaphoreType.DMA,
        pltpu.SemaphoreType.DMA,
    ],
)
def _sc_gather_deg(hp_hbm, src_hbm, dst_hbm, out_hbm, deg_hbm, h_sh, acc_sh,
                   idx_v, didx_v, gb_v, ones_v, nar_v, sem, sem2, sem3):
    c = lax.axis_index("c")
    s = lax.axis_index("s")
    w = c * NS + s

    def zloc(i, carry):
        nar_v[i, :] = jnp.zeros((H,), jnp.float32)
        return carry

    lax.fori_loop(0, 128, zloc, 0)

    def zcp(p, carry):
        pltpu.sync_copy(nar_v, acc_sh.at[pl.ds(s * STRIPE + p * 128, 128)])
        return carry

    lax.fori_loop(0, STRIPE // 128, zcp, 0)

    def oloc(i, carry):
        ones_v[i, :] = jnp.ones((H,), jnp.float32)
        return carry

    lax.fori_loop(0, CH, oloc, 0)
    pltpu.sync_copy(hp_hbm.at[pl.ds(s * STRIPE, STRIPE)],
                    h_sh.at[pl.ds(s * STRIPE, STRIPE)])
    plsc.subcore_barrier()

    @pl.when(w < NW)
    def _():
        pltpu.sync_copy(src_hbm.at[w], idx_v)
        pltpu.sync_copy(dst_hbm.at[w], didx_v)

        def step(g, carry):
            @pl.when(g < NG)
            def _():
                for u in range(FG):
                    pltpu.async_copy(
                        h_sh.at[idx_v.at[g * FG + u]],
                        gb_v.at[g % 2].at[u], sem)
                    pltpu.async_copy(
                        ones_v, acc_sh.at[didx_v.at[g * FG + u]], sem3,
                        add=True)

            @pl.when(g > 0)
            def _():
                p = g - 1
                for u in range(FG):
                    pltpu.make_async_copy(
                        h_sh.at[idx_v.at[p * FG + u]],
                        gb_v.at[p % 2].at[u], sem).wait()
                    pltpu.make_async_copy(
                        ones_v, acc_sh.at[didx_v.at[p * FG + u]],
                        sem3).wait()
                pltpu.async_copy(
                    gb_v.at[p % 2],
                    out_hbm.at[w].at[pl.ds(p * FG, FG)], sem2).wait()

            return carry

        lax.fori_loop(0, NG + 1, step, 0)

    plsc.subcore_barrier()
    pltpu.sync_copy(acc_sh.at[pl.ds(s * STRIPE, STRIPE)],
                    deg_hbm.at[c].at[pl.ds(s * STRIPE, STRIPE)])


@functools.partial(
    pl.kernel,
    out_type=jax.ShapeDtypeStruct((NC, N_PAD, H), jnp.float32),
    mesh=_MESH,
    compiler_params=_SC_PARAMS,
    scratch_types=[
        pltpu.VMEM_SHARED((N_PAD, H), jnp.float32),
        pltpu.VMEM((NCH, CH), jnp.int32),
        pltpu.VMEM((2, FG, CH, H), jnp.float32),
        pltpu.VMEM((128, H), jnp.float32),
        pltpu.SemaphoreType.DMA,
        pltpu.SemaphoreType.DMA,
    ],
)
def _sc_scatter_add(msg_hbm, dst_hbm, out_hbm, acc_sh, idx_v, gb_v, nar_v,
                    sem, sem2):
    c = lax.axis_index("c")
    s = lax.axis_index("s")
    w = c * NS + s
    def zloc(i, carry):
        nar_v[i, :] = jnp.zeros((H,), jnp.float32)
        return carry

    lax.fori_loop(0, 128, zloc, 0)

    def zcp(p, carry):
        pltpu.sync_copy(nar_v, acc_sh.at[pl.ds(s * STRIPE + p * 128, 128)])
        return carry

    lax.fori_loop(0, STRIPE // 128, zcp, 0)
    plsc.subcore_barrier()

    @pl.when(w < NW)
    def _():
        pltpu.sync_copy(dst_hbm.at[w], idx_v)

        def step(g, carry):
            @pl.when(g < NG)
            def _():
                pltpu.async_copy(
                    msg_hbm.at[w].at[pl.ds(g * FG, FG)],
                    gb_v.at[g % 2], sem2).wait()
                for u in range(FG):
                    pltpu.async_copy(
                        gb_v.at[g % 2].at[u],
                        acc_sh.at[idx_v.at[g * FG + u]], sem, add=True)

            @pl.when(g > 0)
            def _():
                p = g - 1
                for u in range(FG):
                    pltpu.make_async_copy(
                        gb_v.at[p % 2].at[u],
                        acc_sh.at[idx_v.at[p * FG + u]], sem).wait()

            return carry

        lax.fori_loop(0, NG + 1, step, 0)

    plsc.subcore_barrier()
    pltpu.sync_copy(acc_sh.at[pl.ds(s * STRIPE, STRIPE)],
                    out_hbm.at[c].at[pl.ds(s * STRIPE, STRIPE)])


def _prep_body(x_ref, aw_ref, ab_ref, h0_ref):
    h0_ref[...] = jnp.maximum(
        jnp.dot(x_ref[...], aw_ref[...],
                preferred_element_type=jnp.float32) + ab_ref[...], 0.0)


def _tc_prep(x8, aw8, ab8):
    return pl.pallas_call(
        _prep_body,
        out_shape=jax.ShapeDtypeStruct((NRP, 128), jnp.float32),
    )(x8, aw8, ab8)


EB8 = 400


def _msg_body(ea_ref, hs_ref, e1w_ref, e1b_ref, e2w_ref, e2b_ref, r_ref,
              s_ref, msg_ref):
    t = _gelu(jnp.dot(ea_ref[...], e1w_ref[...],
                      preferred_element_type=jnp.float32) + e1b_ref[...])
    we = jnp.dot(t, e2w_ref[...],
                 preferred_element_type=jnp.float32) + e2b_ref[...]
    hexp = jnp.dot(hs_ref[...], r_ref[...], preferred_element_type=jnp.float32)
    msg_ref[...] = jnp.dot(we * hexp, s_ref[...],
                           preferred_element_type=jnp.float32)


def _tc_msg(ea8, hs8, e1w8, e1b8, e2w8, e2b8, r8, s8):
    grid = (E8 // EB8,)
    return pl.pallas_call(
        _msg_body,
        grid=grid,
        in_specs=[
            pl.BlockSpec((EB8, 128), lambda i: (i, 0)),
            pl.BlockSpec((EB8, 128), lambda i: (i, 0)),
            pl.BlockSpec((128, 128), lambda i: (0, 0)),
            pl.BlockSpec((1, 128), lambda i: (0, 0)),
            pl.BlockSpec((128, 2048), lambda i: (0, 0)),
            pl.BlockSpec((1, 2048), lambda i: (0, 0)),
            pl.BlockSpec((128, 2048), lambda i: (0, 0)),
            pl.BlockSpec((2048, 128), lambda i: (0, 0)),
        ],
        out_specs=pl.BlockSpec((EB8, 128), lambda i: (i, 0)),
        out_shape=jax.ShapeDtypeStruct((E8, 128), jnp.float32),
    )(ea8, hs8, e1w8, e1b8, e2w8, e2b8, r8, s8)


def _update_body(h_ref, aggp_ref, degp_ref, rw_ref, rb_ref, m_ref, g_ref,
                 b_ref, out_ref):
    h = h_ref[...]
    invd = 1.0 / jnp.maximum(degp_ref[0] + degp_ref[1], 1.0)
    agg = (aggp_ref[0] + aggp_ref[1]) * invd
    hn = jnp.maximum(
        jnp.dot(h, rw_ref[...], preferred_element_type=jnp.float32)
        + rb_ref[...] + agg, 0.0)
    z = h + hn
    mu = jnp.dot(z, m_ref[...], preferred_element_type=jnp.float32)
    zc = z - mu
    var = jnp.dot(zc * zc, m_ref[...], preferred_element_type=jnp.float32)
    out_ref[...] = zc * lax.rsqrt(var + 1e-5) * g_ref[...] + b_ref[...]


def _tc_update(h8, aggp, degp, rw8, rb8, m8, g8, b8):
    return pl.pallas_call(
        _update_body,
        out_shape=jax.ShapeDtypeStruct((NRP, 128), jnp.float32),
    )(h8, aggp, degp, rw8, rb8, m8, g8, b8)


def _pool_body(h_ref, bat_ref, h1w_ref, h1b_ref, h2w_ref, h2b_ref, out_ref):
    h = h_ref[...]
    bat = bat_ref[...]
    gid = lax.broadcasted_iota(jnp.int32, (1, G), 1)
    mol = jnp.zeros((G, H), jnp.float32)
    cnt = jnp.zeros((G, 1), jnp.float32)
    for j in range(8):
        oh = (bat[:, j:j + 1] == gid).astype(jnp.float32)
        mol = mol + lax.dot_general(oh, h[:, j * H:(j + 1) * H],
                                    (((0,), (0,)), ((), ())),
                                    preferred_element_type=jnp.float32)
        cnt = cnt + lax.dot_general(oh, jnp.ones((NRP, 1), jnp.float32),
                                    (((0,), (0,)), ((), ())),
                                    preferred_element_type=jnp.float32)
    mol = mol / jnp.maximum(cnt, 1.0)
    hid = _gelu(jnp.dot(mol, h1w_ref[...],
                        preferred_element_type=jnp.float32) + h1b_ref[...])
    out_ref[...] = jnp.dot(hid, h2w_ref[...],
                           preferred_element_type=jnp.float32) + h2b_ref[...]


def _tc_pool(h8, bat8, h1w, h1b, h2w, h2b):
    return pl.pallas_call(
        _pool_body,
        out_shape=jax.ShapeDtypeStruct((G, 1), jnp.float32),
    )(h8, bat8, h1w, h1b, h2w, h2b)


def _bd8(w):
    return jnp.einsum("ij,ab->iajb", jnp.eye(8, dtype=w.dtype),
                      w).reshape(8 * w.shape[0], 8 * w.shape[1])


def _tile8(b):
    return jnp.tile(b, 8)[None, :]


def kernel(x, edge_index, edge_attr, batch, atom_W, atom_b, e1_W, e1_b, e2_W,
           e2_b, root_W, root_b, ln_g, ln_b, h1_W, h1_b, h2_W, h2_b):
    src3 = edge_index[0].reshape(NW, NCH, CH)
    dst3 = edge_index[1].reshape(NW, NCH, CH)
    ea8 = edge_attr.reshape(E8, 128)
    x8 = jnp.concatenate(
        [x.reshape(NR, 8 * AF),
         jnp.zeros((NRP - NR, 8 * AF), jnp.float32)])
    bat8 = jnp.concatenate(
        [batch, jnp.full((N_PAD - N,), G, jnp.int32)]).reshape(NRP, 8)

    ii = jnp.arange(H * H)
    rmat = (jnp.arange(H)[:, None] == (ii[None, :] // H)).astype(jnp.float32)
    smat = ((ii[:, None] % H) == jnp.arange(H)[None, :]).astype(jnp.float32)
    r8 = _bd8(rmat)
    s8 = _bd8(smat)
    m8 = _bd8(jnp.full((H, H), 1.0 / H, jnp.float32))

    h8 = _tc_prep(x8, _bd8(atom_W), _tile8(atom_b))
    degp = None

    for l in range(L):
        if l == 0:
            hs4, degp4 = _sc_gather_deg(h8.reshape(N_PAD, H), src3, dst3)
            hs8 = hs4.reshape(E8, 128)
            degp = degp4.reshape(NC, NRP, 128)
        else:
            hs8 = _sc_gather(h8.reshape(N_PAD, H), src3).reshape(E8, 128)
        msg8 = _tc_msg(ea8, hs8, _bd8(e1_W[l]), _tile8(e1_b[l]),
                       _bd8(e2_W[l]), _tile8(e2_b[l]), r8, s8)
        aggp = _sc_scatter_add(msg8.reshape(NW, NCH, CH, H),
                               dst3).reshape(NC, NRP, 128)
        h8 = _tc_update(h8, aggp, degp, _bd8(root_W[l]), _tile8(root_b[l]),
                        m8, _tile8(ln_g[l]), _tile8(ln_b[l]))

    logits = _tc_pool(h8, bat8, h1_W, h1_b.reshape(1, H // 2), h2_W,
                      h2_b.reshape(1, 1))
    return logits.reshape(G)


# speedup vs baseline: 1.1033x; 1.1033x over previous
import functools
import math

import jax
import jax.numpy as jnp
from jax import lax
from jax.experimental import pallas as pl
from jax.experimental.pallas import tpu as pltpu
from jax.experimental.pallas import tpu_sc as plsc

N = 10000
E = 160000
G = 256
H = 16
AF = 128
BF = 16
L = 3

NR = N // 8
E8 = E // 8
N_PAD = 10240
NRP = N_PAD // 8
STRIPE = N_PAD // 16
PSTRIPE = NRP // 16

NC = 2
NS = 16
NW = 25
EW = E // NW
CH = 128
NCH = EW // CH
FG = 5
NG = NCH // FG

_SQRT2 = math.sqrt(2.0)


def _gelu(x):
    return 0.5 * x * (1.0 + lax.erf(x / _SQRT2))


_MESH = plsc.VectorSubcoreMesh(core_axis_name="c", subcore_axis_name="s",
                               num_cores=NC, num_subcores=NS)
_SC_PARAMS = pltpu.CompilerParams(use_tc_tiling_on_sc=False)


@functools.partial(
    pl.kernel,
    out_type=jax.ShapeDtypeStruct((NW, NCH, CH, H), jnp.float32),
    mesh=_MESH,
    compiler_params=_SC_PARAMS,
    scratch_types=[
        pltpu.VMEM_SHARED((N_PAD, H), jnp.float32),
        pltpu.VMEM((NCH, CH), jnp.int32),
        pltpu.VMEM((2, FG, CH, H), jnp.float32),
        pltpu.SemaphoreType.DMA,
        pltpu.SemaphoreType.DMA,
    ],
)
def _sc_gather(hp_hbm, src_hbm, out_hbm, h_sh, idx_v, gb_v, sem, sem2):
    c = lax.axis_index("c")
    s = lax.axis_index("s")
    w = c * NS + s
    pltpu.sync_copy(hp_hbm.at[pl.ds(s * STRIPE, STRIPE)],
                    h_sh.at[pl.ds(s * STRIPE, STRIPE)])
    plsc.subcore_barrier()

    @pl.when(w < NW)
    def _():
        pltpu.sync_copy(src_hbm.at[w], idx_v)

        def step(g, carry):
            @pl.when(g < NG)
            def _():
                for u in range(FG):
                    pltpu.async_copy(
                        h_sh.at[idx_v.at[g * FG + u]],
                        gb_v.at[g % 2].at[u], sem)

            @pl.when(g > 0)
            def _():
                p = g - 1
                for u in range(FG):
                    pltpu.make_async_copy(
                        h_sh.at[idx_v.at[p * FG + u]],
                        gb_v.at[p % 2].at[u], sem).wait()
                pltpu.async_copy(
                    gb_v.at[p % 2],
                    out_hbm.at[w].at[pl.ds(p * FG, FG)], sem2).wait()

            return carry

        lax.fori_loop(0, NG + 1, step, 0)


@functools.partial(
    pl.kernel,
    out_type=(jax.ShapeDtypeStruct((NW, NCH, CH, H), jnp.float32),
              jax.ShapeDtypeStruct((NC, N_PAD, H), jnp.float32)),
    mesh=_MESH,
    compiler_params=_SC_PARAMS,
    scratch_types=[
        pltpu.VMEM_SHARED((N_PAD, H), jnp.float32),
        pltpu.VMEM_SHARED((N_PAD, H), jnp.float32),
        pltpu.VMEM((NCH, CH), jnp.int32),
        pltpu.VMEM((NCH, CH), jnp.int32),
        pltpu.VMEM((2, FG, CH, H), jnp.float32),
        pltpu.VMEM((CH, H), jnp.float32),
        pltpu.VMEM((128, H), jnp.float32),
        pltpu.SemaphoreType.DMA,
        pltpu.SemaphoreType.DMA,
        pltpu.SemaphoreType.DMA,
    ],
)
def _sc_gather_deg(hp_hbm, src_hbm, dst_hbm, out_hbm, deg_hbm, h_sh, acc_sh,
                   idx_v, didx_v, gb_v, ones_v, nar_v, sem, sem2, sem3):
    c = lax.axis_index("c")
    s = lax.axis_index("s")
    w = c * NS + s

    def zloc(i, carry):
        nar_v[i, :] = jnp.zeros((H,), jnp.float32)
        return carry

    lax.fori_loop(0, 128, zloc, 0)

    def zcp(p, carry):
        pltpu.sync_copy(nar_v, acc_sh.at[pl.ds(s * STRIPE + p * 128, 128)])
        return carry

    lax.fori_loop(0, STRIPE // 128, zcp, 0)

    def oloc(i, carry):
        ones_v[i, :] = jnp.ones((H,), jnp.float32)
        return carry

    lax.fori_loop(0, CH, oloc, 0)
    pltpu.sync_copy(hp_hbm.at[pl.ds(s * STRIPE, STRIPE)],
                    h_sh.at[pl.ds(s * STRIPE, STRIPE)])
    plsc.subcore_barrier()

    @pl.when(w < NW)
    def _():
        pltpu.sync_copy(src_hbm.at[w], idx_v)
        pltpu.sync_copy(dst_hbm.at[w], didx_v)

        def step(g, carry):
            @pl.when(g < NG)
            def _():
                for u in range(FG):
                    pltpu.async_copy(
                        h_sh.at[idx_v.at[g * FG + u]],
                        gb_v.at[g % 2].at[u], sem)
                    pltpu.async_copy(
                        ones_v, acc_sh.at[didx_v.at[g * FG + u]], sem3,
                        add=True)

            @pl.when(g > 0)
            def _():
                p = g - 1
                for u in range(FG):
                    pltpu.make_async_copy(
                        h_sh.at[idx_v.at[p * FG + u]],
                        gb_v.at[p % 2].at[u], sem).wait()
                    pltpu.make_async_copy(
                        ones_v, acc_sh.at[didx_v.at[p * FG + u]],
                        sem3).wait()
                pltpu.async_copy(
                    gb_v.at[p % 2],
                    out_hbm.at[w].at[pl.ds(p * FG, FG)], sem2).wait()

            return carry

        lax.fori_loop(0, NG + 1, step, 0)

    plsc.subcore_barrier()
    pltpu.sync_copy(acc_sh.at[pl.ds(s * STRIPE, STRIPE)],
                    deg_hbm.at[c].at[pl.ds(s * STRIPE, STRIPE)])


@functools.partial(
    pl.kernel,
    out_type=jax.ShapeDtypeStruct((NC, N_PAD, H), jnp.float32),
    mesh=_MESH,
    compiler_params=_SC_PARAMS,
    scratch_types=[
        pltpu.VMEM_SHARED((N_PAD, H), jnp.float32),
        pltpu.VMEM((NCH, CH), jnp.int32),
        pltpu.VMEM((2, FG, CH, H), jnp.float32),
        pltpu.VMEM((128, H), jnp.float32),
        pltpu.SemaphoreType.DMA,
        pltpu.SemaphoreType.DMA,
    ],
)
def _sc_scatter_add(msg_hbm, dst_hbm, out_hbm, acc_sh, idx_v, gb_v, nar_v,
                    sem, sem2):
    c = lax.axis_index("c")
    s = lax.axis_index("s")
    w = c * NS + s
    def zloc(i, carry):
        nar_v[i, :] = jnp.zeros((H,), jnp.float32)
        return carry

    lax.fori_loop(0, 128, zloc, 0)

    def zcp(p, carry):
        pltpu.sync_copy(nar_v, acc_sh.at[pl.ds(s * STRIPE + p * 128, 128)])
        return carry

    lax.fori_loop(0, STRIPE // 128, zcp, 0)
    plsc.subcore_barrier()

    @pl.when(w < NW)
    def _():
        pltpu.sync_copy(dst_hbm.at[w], idx_v)

        def step(g, carry):
            @pl.when(g < NG)
            def _():
                pltpu.async_copy(
                    msg_hbm.at[w].at[pl.ds(g * FG, FG)],
                    gb_v.at[g % 2], sem2).wait()
                for u in range(FG):
                    pltpu.async_copy(
                        gb_v.at[g % 2].at[u],
                        acc_sh.at[idx_v.at[g * FG + u]], sem, add=True)

            @pl.when(g > 0)
            def _():
                p = g - 1
                for u in range(FG):
                    pltpu.make_async_copy(
                        gb_v.at[p % 2].at[u],
                        acc_sh.at[idx_v.at[p * FG + u]], sem).wait()

            return carry

        lax.fori_loop(0, NG + 1, step, 0)

    plsc.subcore_barrier()
    pltpu.sync_copy(acc_sh.at[pl.ds(s * STRIPE, STRIPE)],
                    out_hbm.at[c].at[pl.ds(s * STRIPE, STRIPE)])


def _prep_body(x_ref, aw_ref, ab_ref, h0_ref):
    h0_ref[...] = jnp.maximum(
        jnp.dot(x_ref[...], aw_ref[...],
                preferred_element_type=jnp.float32) + ab_ref[...], 0.0)


def _tc_prep(x8, aw8, ab8):
    return pl.pallas_call(
        _prep_body,
        out_shape=jax.ShapeDtypeStruct((NRP, 128), jnp.float32),
    )(x8, aw8, ab8)


EB8 = 2000


def _msg_body(ea_ref, hs_ref, e1w_ref, e1b_ref, e2w_ref, e2b_ref, r_ref,
              s_ref, msg_ref):
    t = _gelu(jnp.dot(ea_ref[...], e1w_ref[...],
                      preferred_element_type=jnp.float32) + e1b_ref[...])
    we = jnp.dot(t, e2w_ref[...],
                 preferred_element_type=jnp.float32) + e2b_ref[...]
    hexp = jnp.dot(hs_ref[...], r_ref[...], preferred_element_type=jnp.float32)
    msg_ref[...] = jnp.dot(we * hexp, s_ref[...],
                           preferred_element_type=jnp.float32)


def _tc_msg(ea8, hs8, e1w8, e1b8, e2w8, e2b8, r8, s8):
    grid = (E8 // EB8,)
    return pl.pallas_call(
        _msg_body,
        grid=grid,
        in_specs=[
            pl.BlockSpec((EB8, 128), lambda i: (i, 0)),
            pl.BlockSpec((EB8, 128), lambda i: (i, 0)),
            pl.BlockSpec((128, 128), lambda i: (0, 0)),
            pl.BlockSpec((1, 128), lambda i: (0, 0)),
            pl.BlockSpec((128, 2048), lambda i: (0, 0)),
            pl.BlockSpec((1, 2048), lambda i: (0, 0)),
            pl.BlockSpec((128, 2048), lambda i: (0, 0)),
            pl.BlockSpec((2048, 128), lambda i: (0, 0)),
        ],
        out_specs=pl.BlockSpec((EB8, 128), lambda i: (i, 0)),
        out_shape=jax.ShapeDtypeStruct((E8, 128), jnp.float32),
    )(ea8, hs8, e1w8, e1b8, e2w8, e2b8, r8, s8)


def _update_body(h_ref, aggp_ref, degp_ref, rw_ref, rb_ref, m_ref, g_ref,
                 b_ref, out_ref):
    h = h_ref[...]
    invd = 1.0 / jnp.maximum(degp_ref[0] + degp_ref[1], 1.0)
    agg = (aggp_ref[0] + aggp_ref[1]) * invd
    hn = jnp.maximum(
        jnp.dot(h, rw_ref[...], preferred_element_type=jnp.float32)
        + rb_ref[...] + agg, 0.0)
    z = h + hn
    mu = jnp.dot(z, m_ref[...], preferred_element_type=jnp.float32)
    zc = z - mu
    var = jnp.dot(zc * zc, m_ref[...], preferred_element_type=jnp.float32)
    out_ref[...] = zc * lax.rsqrt(var + 1e-5) * g_ref[...] + b_ref[...]


def _tc_update(h8, aggp, degp, rw8, rb8, m8, g8, b8):
    return pl.pallas_call(
        _update_body,
        out_shape=jax.ShapeDtypeStruct((NRP, 128), jnp.float32),
    )(h8, aggp, degp, rw8, rb8, m8, g8, b8)


def _pool_body(h_ref, bat_ref, h1w_ref, h1b_ref, h2w_ref, h2b_ref, out_ref):
    h = h_ref[...]
    bat = bat_ref[...]
    gid = lax.broadcasted_iota(jnp.int32, (1, G), 1)
    mol = jnp.zeros((G, H), jnp.float32)
    cnt = jnp.zeros((G, 1), jnp.float32)
    for j in range(8):
        oh = (bat[:, j:j + 1] == gid).astype(jnp.float32)
        mol = mol + lax.dot_general(oh, h[:, j * H:(j + 1) * H],
                                    (((0,), (0,)), ((), ())),
                                    preferred_element_type=jnp.float32)
        cnt = cnt + lax.dot_general(oh, jnp.ones((NRP, 1), jnp.float32),
                                    (((0,), (0,)), ((), ())),
                                    preferred_element_type=jnp.float32)
    mol = mol / jnp.maximum(cnt, 1.0)
    hid = _gelu(jnp.dot(mol, h1w_ref[...],
                        preferred_element_type=jnp.float32) + h1b_ref[...])
    out_ref[...] = jnp.dot(hid, h2w_ref[...],
                           preferred_element_type=jnp.float32) + h2b_ref[...]


def _tc_pool(h8, bat8, h1w, h1b, h2w, h2b):
    return pl.pallas_call(
        _pool_body,
        out_shape=jax.ShapeDtypeStruct((G, 1), jnp.float32),
    )(h8, bat8, h1w, h1b, h2w, h2b)


def _bd8(w):
    return jnp.einsum("ij,ab->iajb", jnp.eye(8, dtype=w.dtype),
                      w).reshape(8 * w.shape[0], 8 * w.shape[1])


def _tile8(b):
    return jnp.tile(b, 8)[None, :]


def kernel(x, edge_index, edge_attr, batch, atom_W, atom_b, e1_W, e1_b, e2_W,
           e2_b, root_W, root_b, ln_g, ln_b, h1_W, h1_b, h2_W, h2_b):
    src3 = edge_index[0].reshape(NW, NCH, CH)
    dst3 = edge_index[1].reshape(NW, NCH, CH)
    ea8 = edge_attr.reshape(E8, 128)
    x8 = jnp.concatenate(
        [x.reshape(NR, 8 * AF),
         jnp.zeros((NRP - NR, 8 * AF), jnp.float32)])
    bat8 = jnp.concatenate(
        [batch, jnp.full((N_PAD - N,), G, jnp.int32)]).reshape(NRP, 8)

    ii = jnp.arange(H * H)
    rmat = (jnp.arange(H)[:, None] == (ii[None, :] // H)).astype(jnp.float32)
    smat = ((ii[:, None] % H) == jnp.arange(H)[None, :]).astype(jnp.float32)
    r8 = _bd8(rmat)
    s8 = _bd8(smat)
    m8 = _bd8(jnp.full((H, H), 1.0 / H, jnp.float32))

    h8 = _tc_prep(x8, _bd8(atom_W), _tile8(atom_b))
    degp = None

    for l in range(L):
        if l == 0:
            hs4, degp4 = _sc_gather_deg(h8.reshape(N_PAD, H), src3, dst3)
            hs8 = hs4.reshape(E8, 128)
            degp = degp4.reshape(NC, NRP, 128)
        else:
            hs8 = _sc_gather(h8.reshape(N_PAD, H), src3).reshape(E8, 128)
        msg8 = _tc_msg(ea8, hs8, _bd8(e1_W[l]), _tile8(e1_b[l]),
                       _bd8(e2_W[l]), _tile8(e2_b[l]), r8, s8)
        aggp = _sc_scatter_add(msg8.reshape(NW, NCH, CH, H),
                               dst3).reshape(NC, NRP, 128)
        h8 = _tc_update(h8, aggp, degp, _bd8(root_W[l]), _tile8(root_b[l]),
                        m8, _tile8(ln_g[l]), _tile8(ln_b[l]))

    logits = _tc_pool(h8, bat8, h1_W, h1_b.reshape(1, H // 2), h2_W,
                      h2_b.reshape(1, 1))
    return logits.reshape(G)
